# baseline (speedup 1.0000x reference)
import math

import jax
import jax.numpy as jnp
from jax import lax
from jax.experimental import pallas as pl
from jax.experimental.pallas import tpu as pltpu
from jax.experimental.pallas import tpu_sc as plsc

N = 10000
E = 320000
D = 128
H = 4
DH = 32
NT = 4
ET = 8

NP = 10240
NC = 2
NS = 16
NW = NC * NS
EW = E // NW
CB = 80
NCH = EW // CB
NGB = CB // 16
RPT = NP // NS
AW = NP * H
AE_W = H * CB
AE_T = NCH * AE_W
BLK = 25
NBK = NCH // BLK

_F32 = jnp.float32
_I32 = jnp.int32

_SC_PARAMS = pltpu.CompilerParams(needs_layout_passes=False)


def _sc_mesh():
    return plsc.VectorSubcoreMesh(core_axis_name="c", subcore_axis_name="s",
                                  num_cores=NC, num_subcores=NS)


def _k0_body(x_ref, nt_ref, wk_ref, wq_ref, wv_ref, ra_ref, rm_ref,
             q_ref, kp_ref, vp_ref):
    x = x_ref[...]
    nt = nt_ref[...]
    k = jnp.zeros_like(x)
    q = jnp.zeros_like(x)
    v = jnp.zeros_like(x)
    for t in range(NT):
        m = (nt == float(t)).astype(_F32)
        k = k + m * jnp.dot(x, wk_ref[t], preferred_element_type=_F32)
        q = q + m * jnp.dot(x, wq_ref[t], preferred_element_type=_F32)
        v = v + m * jnp.dot(x, wv_ref[t], preferred_element_type=_F32)
    q_ref[...] = q
    for e in range(ET):
        kp_ref[e] = jnp.dot(k, ra_ref[e], preferred_element_type=_F32)
        vp_ref[e] = jnp.dot(v, rm_ref[e], preferred_element_type=_F32)


def _k0(xp, ntf, Wk, Wq, Wv, Ratt, Rmsg):
    B = 512
    g = NP // B
    return pl.pallas_call(
        _k0_body,
        grid=(g,),
        in_specs=[
            pl.BlockSpec((B, D), lambda i: (i, 0)),
            pl.BlockSpec((B, 1), lambda i: (i, 0)),
            pl.BlockSpec((NT, D, D), lambda i: (0, 0, 0)),
            pl.BlockSpec((NT, D, D), lambda i: (0, 0, 0)),
            pl.BlockSpec((NT, D, D), lambda i: (0, 0, 0)),
            pl.BlockSpec((ET, D, D), lambda i: (0, 0, 0)),
            pl.BlockSpec((ET, D, D), lambda i: (0, 0, 0)),
        ],
        out_specs=[
            pl.BlockSpec((B, D), lambda i: (i, 0)),
            pl.BlockSpec((ET, B, D), lambda i: (0, i, 0)),
            pl.BlockSpec((ET, B, D), lambda i: (0, i, 0)),
        ],
        out_shape=[
            jax.ShapeDtypeStruct((NP, D), _F32),
            jax.ShapeDtypeStruct((ET, NP, D), _F32),
            jax.ShapeDtypeStruct((ET, NP, D), _F32),
        ],
    )(xp, ntf, Wk, Wq, Wv, Ratt, Rmsg)


def _k1_body(kp_hbm, q_hbm, meta_hbm,
             aexp_hbm, asum_hbm,
             meta_v, kidx_v, kro, qro, aexpT, asum1d,
             sem1, sem2):
    c = lax.axis_index("c")
    s = lax.axis_index("s")
    wid = s * NC + c
    crow = wid * NCH

    zv = jnp.zeros((16,), _F32)

    def zinit(i, carry):
        asum1d[pl.ds(i * 16, 16)] = zv
        return carry
    lax.fori_loop(0, AW // 16, zinit, 0)

    iota16 = lax.iota(_I32, 16)

    def batch(j, carry):
        pltpu.sync_copy(meta_hbm.at[crow + j], meta_v)

        def mkidx(g, carry2):
            sl = pl.ds(g * 16, 16)
            kidx_v[sl] = meta_v[2, sl] * NP + meta_v[0, sl]
            return carry2
        lax.fori_loop(0, NGB, mkidx, 0, unroll=True)
        cp1 = pltpu.async_copy(kp_hbm.at[kidx_v], kro, sem1)
        cp2 = pltpu.async_copy(q_hbm.at[meta_v.at[1]], qro, sem2)
        cp1.wait()
        cp2.wait()

        def grp(g, carry2):
            rows = g * 16 + iota16
            dst16 = meta_v[1, pl.ds(g * 16, 16)]
            for h in range(H):
                accs = [jnp.zeros((16,), _F32) for _ in range(4)]
                for i, d in enumerate(range(h * DH, (h + 1) * DH)):
                    dd = jnp.full((16,), d, _I32)
                    accs[i % 4] = accs[i % 4] + (
                        plsc.load_gather(kro, [rows, dd])
                        * plsc.load_gather(qro, [rows, dd]))
                ae = jnp.exp((accs[0] + accs[1]) + (accs[2] + accs[3]))
                aexpT[pl.ds(j * AE_W + h * CB + g * 16, 16)] = ae
                plsc.addupdate_scatter(asum1d, [dst16 * H + h], ae)
            return carry2
        lax.fori_loop(0, NGB, grp, 0)
        return carry

    lax.fori_loop(0, NCH, batch, 0)
    pltpu.sync_copy(aexpT, aexp_hbm.at[pl.ds(wid * AE_T, AE_T)])
    pltpu.sync_copy(asum1d, asum_hbm.at[wid])


def _k1(kp2, qp, meta):
    f = pl.kernel(
        _k1_body,
        out_type=[
            jax.ShapeDtypeStruct((NW * AE_T,), _F32),
            jax.ShapeDtypeStruct((NW, AW), _F32),
        ],
        mesh=_sc_mesh(),
        compiler_params=_SC_PARAMS,
        scratch_types=[
            pltpu.VMEM((3, CB), _I32),
            pltpu.VMEM((CB,), _I32),
            pltpu.VMEM((CB, D), _F32),
            pltpu.VMEM((CB, D), _F32),
            pltpu.VMEM((AE_T,), _F32),
            pltpu.VMEM((AW,), _F32),
            pltpu.SemaphoreType.DMA,
            pltpu.SemaphoreType.DMA,
        ],
    )
    return f(kp2, qp, meta)


def _k3_body(vp_hbm, meta_hbm, aexp_hbm, zer128_hbm,
             hacc_hbm,
             meta_v, vidx_v, vro, mrow, aexpB, hacc_sh,
             sem1, sem2):
    c = lax.axis_index("c")
    s = lax.axis_index("s")
    wid = s * NC + c
    row0 = s * RPT
    crow = wid * NCH
    pltpu.sync_copy(zer128_hbm.at[pl.ds(row0, RPT)],
                    hacc_sh.at[pl.ds(row0, RPT)])
    plsc.subcore_barrier()

    iota16 = lax.iota(_I32, 16)

    def block(b, carry0):
        pltpu.sync_copy(
            aexp_hbm.at[pl.ds(wid * AE_T + b * BLK * AE_W, BLK * AE_W)],
            aexpB)

        def batch(jj, carry):
            j = b * BLK + jj
            pltpu.sync_copy(meta_hbm.at[crow + j], meta_v)

            def mkidx(g, carry2):
                sl = pl.ds(g * 16, 16)
                vidx_v[sl] = meta_v[2, sl] * NP + meta_v[0, sl]
                return carry2
            lax.fori_loop(0, NGB, mkidx, 0, unroll=True)
            cp1 = pltpu.async_copy(vp_hbm.at[vidx_v], vro, sem1)
            cp1.wait()

            def grp(g, carry2):
                aes = [aexpB[pl.ds(jj * AE_W + h * CB + g * 16, 16)]
                       for h in range(H)]
                for e in range(16):
                    r = g * 16 + e
                    lane = jnp.full((16,), e, _I32)
                    for h in range(H):
                        sp = aes[h].at[lane].get(mode="promise_in_bounds")
                        lo = h * DH
                        mrow[r, pl.ds(lo, 16)] = vro[r, pl.ds(lo, 16)] * sp
                        mrow[r, pl.ds(lo + 16, 16)] = (
                            vro[r, pl.ds(lo + 16, 16)] * sp)
                return carry2
            lax.fori_loop(0, NGB, grp, 0)
            pltpu.sync_copy(mrow, hacc_sh.at[meta_v.at[1]], add=True)
            return carry

        lax.fori_loop(0, BLK, batch, 0)
        return carry0

    lax.fori_loop(0, NBK, block, 0)
    plsc.subcore_barrier()
    pltpu.sync_copy(hacc_sh.at[pl.ds(row0, RPT)],
                    hacc_hbm.at[c, pl.ds(row0, RPT)])


def _k3(vp2, meta, aexp, zer128):
    f = pl.kernel(
        _k3_body,
        out_type=jax.ShapeDtypeStruct((NC, NP, D), _F32),
        mesh=_sc_mesh(),
        compiler_params=_SC_PARAMS,
        scratch_types=[
            pltpu.VMEM((3, CB), _I32),
            pltpu.VMEM((CB,), _I32),
            pltpu.VMEM((CB, D), _F32),
            pltpu.VMEM((CB, D), _F32),
            pltpu.VMEM((BLK * AE_W,), _F32),
            pltpu.VMEM_SHARED((NP, D), _F32),
            pltpu.SemaphoreType.DMA,
            pltpu.SemaphoreType.DMA,
        ],
    )
    return f(vp2, meta, aexp, zer128)


def _k4_body(h0_ref, h1_ref, asum_ref, x_ref, nt_ref, wa_ref, sk_ref,
             out_ref):
    hs = h0_ref[...] + h1_ref[...]
    x = x_ref[...]
    nt = nt_ref[...]
    den4 = jnp.maximum(jnp.sum(asum_ref[...], axis=0), 1e-30)
    den = jnp.concatenate(
        [jnp.broadcast_to(den4[:, h:h + 1], (x.shape[0], DH))
         for h in range(H)], axis=1)
    hn = hs / den
    sig = 1.0 / (1.0 + jnp.exp(-sk_ref[...]))
    acc = jnp.zeros_like(x)
    alpha = jnp.zeros_like(nt)
    for t in range(NT):
        m = (nt == float(t)).astype(_F32)
        acc = acc + m * jnp.dot(hn, wa_ref[t], preferred_element_type=_F32)
        alpha = alpha + m * jnp.broadcast_to(sig[0:1, t:t + 1], nt.shape)
    out_ref[...] = acc * alpha + x * (1.0 - alpha)


def _k4(h0, h1, asum3, xp, ntf, Wa, sk8):
    B = 512
    g = NP // B
    return pl.pallas_call(
        _k4_body,
        grid=(g,),
        in_specs=[
            pl.BlockSpec((B, D), lambda i: (i, 0)),
            pl.BlockSpec((B, D), lambda i: (i, 0)),
            pl.BlockSpec((NW, B, H), lambda i: (0, i, 0)),
            pl.BlockSpec((B, D), lambda i: (i, 0)),
            pl.BlockSpec((B, 1), lambda i: (i, 0)),
            pl.BlockSpec((NT, D, D), lambda i: (0, 0, 0)),
            pl.BlockSpec((8, NT), lambda i: (0, 0)),
        ],
        out_specs=pl.BlockSpec((B, D), lambda i: (i, 0)),
        out_shape=jax.ShapeDtypeStruct((NP, D), _F32),
    )(h0, h1, asum3, xp, ntf, Wa, sk8)


def kernel(x_node, edge_index, ntype, etype, Wk, Wq, Wv, Wa,
           rel_att, rel_msg, rel_pri, skip):
    xp = jnp.zeros((NP, D), _F32).at[:N].set(x_node)
    ntf = jnp.zeros((NP, 1), _F32).at[:N, 0].set(ntype.astype(_F32))

    scale = rel_pri / math.sqrt(DH)
    Ratt = jnp.zeros((ET, D, D), _F32)
    Rmsg = jnp.zeros((ET, D, D), _F32)
    for i in range(H):
        blk = slice(i * DH, (i + 1) * DH)
        Ratt = Ratt.at[:, blk, blk].set(rel_att[i] * scale[i][:, None, None])
        Rmsg = Rmsg.at[:, blk, blk].set(rel_msg[i])

    meta = jnp.stack([edge_index[0].reshape(E // CB, CB),
                      edge_index[1].reshape(E // CB, CB),
                      etype.reshape(E // CB, CB)], axis=1)
    zer128 = jnp.zeros((NP, D), _F32)
    sk8 = jnp.broadcast_to(skip.reshape(1, NT), (8, NT))

    qp, kp, vp = _k0(xp, ntf, Wk, Wq, Wv, Ratt, Rmsg)
    kp2 = kp.reshape(ET * NP, D)
    vp2 = vp.reshape(ET * NP, D)

    aexp, asum_parts = _k1(kp2, qp, meta)
    hacc = _k3(vp2, meta, aexp, zer128)
    asum3 = asum_parts.reshape(NW, NP, H)
    out = _k4(hacc[0], hacc[1], asum3, xp, ntf, Wa, sk8)
    return out[:N]


# speedup vs baseline: 2.1540x; 2.1540x over previous
import math

import jax
import jax.numpy as jnp
from jax import lax
from jax.experimental import pallas as pl
from jax.experimental.pallas import tpu as pltpu
from jax.experimental.pallas import tpu_sc as plsc

N = 10000
E = 320000
D = 128
H = 4
DH = 32
NT = 4
ET = 8

NP = 10240
NC = 2
NS = 16
NW = NC * NS
EW = E // NW
CB = 80
NCH = EW // CB
NGB = CB // 16
RPT = NP // NS
AW = NP * H
AE_W = H * CB
AE_T = NCH * AE_W
BLK = 25
NBK = NCH // BLK

_F32 = jnp.float32
_I32 = jnp.int32

_SC_PARAMS = pltpu.CompilerParams(needs_layout_passes=False)


def _sc_mesh():
    return plsc.VectorSubcoreMesh(core_axis_name="c", subcore_axis_name="s",
                                  num_cores=NC, num_subcores=NS)


def _k0_body(x_ref, nt_ref, wk_ref, wq_ref, wv_ref, ra_ref, rm_ref,
             q_ref, kp_ref, vp_ref):
    x = x_ref[...]
    nt = nt_ref[...]
    k = jnp.zeros_like(x)
    q = jnp.zeros_like(x)
    v = jnp.zeros_like(x)
    for t in range(NT):
        m = (nt == float(t)).astype(_F32)
        k = k + m * jnp.dot(x, wk_ref[t], preferred_element_type=_F32)
        q = q + m * jnp.dot(x, wq_ref[t], preferred_element_type=_F32)
        v = v + m * jnp.dot(x, wv_ref[t], preferred_element_type=_F32)
    q_ref[...] = q
    for e in range(ET):
        kp_ref[e] = jnp.dot(k, ra_ref[e], preferred_element_type=_F32)
        vp_ref[e] = jnp.dot(v, rm_ref[e], preferred_element_type=_F32)


def _k0(xp, ntf, Wk, Wq, Wv, Ratt, Rmsg):
    B = 512
    g = NP // B
    return pl.pallas_call(
        _k0_body,
        grid=(g,),
        in_specs=[
            pl.BlockSpec((B, D), lambda i: (i, 0)),
            pl.BlockSpec((B, 1), lambda i: (i, 0)),
            pl.BlockSpec((NT, D, D), lambda i: (0, 0, 0)),
            pl.BlockSpec((NT, D, D), lambda i: (0, 0, 0)),
            pl.BlockSpec((NT, D, D), lambda i: (0, 0, 0)),
            pl.BlockSpec((ET, D, D), lambda i: (0, 0, 0)),
            pl.BlockSpec((ET, D, D), lambda i: (0, 0, 0)),
        ],
        out_specs=[
            pl.BlockSpec((B, D), lambda i: (i, 0)),
            pl.BlockSpec((ET, B, D), lambda i: (0, i, 0)),
            pl.BlockSpec((ET, B, D), lambda i: (0, i, 0)),
        ],
        out_shape=[
            jax.ShapeDtypeStruct((NP, D), _F32),
            jax.ShapeDtypeStruct((ET, NP, D), _F32),
            jax.ShapeDtypeStruct((ET, NP, D), _F32),
        ],
    )(xp, ntf, Wk, Wq, Wv, Ratt, Rmsg)


def _k1_body(kp_hbm, q_hbm, meta_hbm,
             aexp_hbm, asum_hbm,
             meta_v, kidx_v, kro, qro, aexpT, asum1d,
             sem1, sem2):
    c = lax.axis_index("c")
    s = lax.axis_index("s")
    wid = s * NC + c
    crow = wid * NCH

    zv = jnp.zeros((16,), _F32)

    def zinit(i, carry):
        asum1d[pl.ds(i * 16, 16)] = zv
        return carry
    lax.fori_loop(0, AW // 16, zinit, 0)

    iota16 = lax.iota(_I32, 16)
    rev4 = (((iota16 & 1) << 3) | ((iota16 & 2) << 1)
            | ((iota16 & 4) >> 1) | ((iota16 & 8) >> 3))

    def batch(j, carry):
        pltpu.sync_copy(meta_hbm.at[crow + j], meta_v)

        def mkidx(g, carry2):
            sl = pl.ds(g * 16, 16)
            kidx_v[sl] = meta_v[2, sl] * NP + meta_v[0, sl]
            return carry2
        lax.fori_loop(0, NGB, mkidx, 0, unroll=True)
        cp1 = pltpu.async_copy(kp_hbm.at[kidx_v], kro, sem1)
        cp2 = pltpu.async_copy(q_hbm.at[meta_v.at[1]], qro, sem2)
        cp1.wait()
        cp2.wait()

        def grp(g, carry2):
            dst16 = meta_v[1, pl.ds(g * 16, 16)]
            for h in range(H):
                lo = h * DH
                ps = []
                for e in range(16):
                    r = g * 16 + e
                    ps.append(
                        kro[r, pl.ds(lo, 16)] * qro[r, pl.ds(lo, 16)]
                        + kro[r, pl.ds(lo + 16, 16)]
                        * qro[r, pl.ds(lo + 16, 16)])
                for k in (8, 4, 2, 1):
                    mask = (iota16 & k) == 0
                    ix = iota16 ^ k
                    nxt = []
                    for i in range(0, len(ps), 2):
                        a, b = ps[i], ps[i + 1]
                        ax = a.at[ix].get(mode="promise_in_bounds")
                        bx = b.at[ix].get(mode="promise_in_bounds")
                        nxt.append(jnp.where(mask, a, b)
                                   + jnp.where(mask, ax, bx))
                    ps = nxt
                s = ps[0].at[rev4].get(mode="promise_in_bounds")
                ae = jnp.exp(s)
                aexpT[pl.ds(j * AE_W + h * CB + g * 16, 16)] = ae
                plsc.addupdate_scatter(asum1d, [dst16 * H + h], ae)
            return carry2
        lax.fori_loop(0, NGB, grp, 0)
        return carry

    lax.fori_loop(0, NCH, batch, 0)
    pltpu.sync_copy(aexpT, aexp_hbm.at[pl.ds(wid * AE_T, AE_T)])
    pltpu.sync_copy(asum1d, asum_hbm.at[wid])


def _k1(kp2, qp, meta):
    f = pl.kernel(
        _k1_body,
        out_type=[
            jax.ShapeDtypeStruct((NW * AE_T,), _F32),
            jax.ShapeDtypeStruct((NW, AW), _F32),
        ],
        mesh=_sc_mesh(),
        compiler_params=_SC_PARAMS,
        scratch_types=[
            pltpu.VMEM((3, CB), _I32),
            pltpu.VMEM((CB,), _I32),
            pltpu.VMEM((CB, D), _F32),
            pltpu.VMEM((CB, D), _F32),
            pltpu.VMEM((AE_T,), _F32),
            pltpu.VMEM((AW,), _F32),
            pltpu.SemaphoreType.DMA,
            pltpu.SemaphoreType.DMA,
        ],
    )
    return f(kp2, qp, meta)


def _k3_body(vp_hbm, meta_hbm, aexp_hbm, zer128_hbm,
             hacc_hbm,
             meta_v, vidx_v, vro, mrow, aexpB, hacc_sh,
             sem1, sem2):
    c = lax.axis_index("c")
    s = lax.axis_index("s")
    wid = s * NC + c
    row0 = s * RPT
    crow = wid * NCH
    pltpu.sync_copy(zer128_hbm.at[pl.ds(row0, RPT)],
                    hacc_sh.at[pl.ds(row0, RPT)])
    plsc.subcore_barrier()

    iota16 = lax.iota(_I32, 16)

    def block(b, carry0):
        pltpu.sync_copy(
            aexp_hbm.at[pl.ds(wid * AE_T + b * BLK * AE_W, BLK * AE_W)],
            aexpB)

        def batch(jj, carry):
            j = b * BLK + jj
            pltpu.sync_copy(meta_hbm.at[crow + j], meta_v)

            def mkidx(g, carry2):
                sl = pl.ds(g * 16, 16)
                vidx_v[sl] = meta_v[2, sl] * NP + meta_v[0, sl]
                return carry2
            lax.fori_loop(0, NGB, mkidx, 0, unroll=True)
            cp1 = pltpu.async_copy(vp_hbm.at[vidx_v], vro, sem1)
            cp1.wait()

            def grp(g, carry2):
                aes = [aexpB[pl.ds(jj * AE_W + h * CB + g * 16, 16)]
                       for h in range(H)]
                for e in range(16):
                    r = g * 16 + e
                    lane = jnp.full((16,), e, _I32)
                    for h in range(H):
                        sp = aes[h].at[lane].get(mode="promise_in_bounds")
                        lo = h * DH
                        mrow[r, pl.ds(lo, 16)] = vro[r, pl.ds(lo, 16)] * sp
                        mrow[r, pl.ds(lo + 16, 16)] = (
                            vro[r, pl.ds(lo + 16, 16)] * sp)
                return carry2
            lax.fori_loop(0, NGB, grp, 0)
            pltpu.sync_copy(mrow, hacc_sh.at[meta_v.at[1]], add=True)
            return carry

        lax.fori_loop(0, BLK, batch, 0)
        return carry0

    lax.fori_loop(0, NBK, block, 0)
    plsc.subcore_barrier()
    pltpu.sync_copy(hacc_sh.at[pl.ds(row0, RPT)],
                    hacc_hbm.at[c, pl.ds(row0, RPT)])


def _k3(vp2, meta, aexp, zer128):
    f = pl.kernel(
        _k3_body,
        out_type=jax.ShapeDtypeStruct((NC, NP, D), _F32),
        mesh=_sc_mesh(),
        compiler_params=_SC_PARAMS,
        scratch_types=[
            pltpu.VMEM((3, CB), _I32),
            pltpu.VMEM((CB,), _I32),
            pltpu.VMEM((CB, D), _F32),
            pltpu.VMEM((CB, D), _F32),
            pltpu.VMEM((BLK * AE_W,), _F32),
            pltpu.VMEM_SHARED((NP, D), _F32),
            pltpu.SemaphoreType.DMA,
            pltpu.SemaphoreType.DMA,
        ],
    )
    return f(vp2, meta, aexp, zer128)


def _k4_body(h0_ref, h1_ref, asum_ref, x_ref, nt_ref, wa_ref, sk_ref,
             out_ref):
    hs = h0_ref[...] + h1_ref[...]
    x = x_ref[...]
    nt = nt_ref[...]
    den4 = jnp.maximum(jnp.sum(asum_ref[...], axis=0), 1e-30)
    den = jnp.concatenate(
        [jnp.broadcast_to(den4[:, h:h + 1], (x.shape[0], DH))
         for h in range(H)], axis=1)
    hn = hs / den
    sig = 1.0 / (1.0 + jnp.exp(-sk_ref[...]))
    acc = jnp.zeros_like(x)
    alpha = jnp.zeros_like(nt)
    for t in range(NT):
        m = (nt == float(t)).astype(_F32)
        acc = acc + m * jnp.dot(hn, wa_ref[t], preferred_element_type=_F32)
        alpha = alpha + m * jnp.broadcast_to(sig[0:1, t:t + 1], nt.shape)
    out_ref[...] = acc * alpha + x * (1.0 - alpha)


def _k4(h0, h1, asum3, xp, ntf, Wa, sk8):
    B = 512
    g = NP // B
    return pl.pallas_call(
        _k4_body,
        grid=(g,),
        in_specs=[
            pl.BlockSpec((B, D), lambda i: (i, 0)),
            pl.BlockSpec((B, D), lambda i: (i, 0)),
            pl.BlockSpec((NW, B, H), lambda i: (0, i, 0)),
            pl.BlockSpec((B, D), lambda i: (i, 0)),
            pl.BlockSpec((B, 1), lambda i: (i, 0)),
            pl.BlockSpec((NT, D, D), lambda i: (0, 0, 0)),
            pl.BlockSpec((8, NT), lambda i: (0, 0)),
        ],
        out_specs=pl.BlockSpec((B, D), lambda i: (i, 0)),
        out_shape=jax.ShapeDtypeStruct((NP, D), _F32),
    )(h0, h1, asum3, xp, ntf, Wa, sk8)


def kernel(x_node, edge_index, ntype, etype, Wk, Wq, Wv, Wa,
           rel_att, rel_msg, rel_pri, skip):
    xp = jnp.zeros((NP, D), _F32).at[:N].set(x_node)
    ntf = jnp.zeros((NP, 1), _F32).at[:N, 0].set(ntype.astype(_F32))

    scale = rel_pri / math.sqrt(DH)
    Ratt = jnp.zeros((ET, D, D), _F32)
    Rmsg = jnp.zeros((ET, D, D), _F32)
    for i in range(H):
        blk = slice(i * DH, (i + 1) * DH)
        Ratt = Ratt.at[:, blk, blk].set(rel_att[i] * scale[i][:, None, None])
        Rmsg = Rmsg.at[:, blk, blk].set(rel_msg[i])

    meta = jnp.stack([edge_index[0].reshape(E // CB, CB),
                      edge_index[1].reshape(E // CB, CB),
                      etype.reshape(E // CB, CB)], axis=1)
    zer128 = jnp.zeros((NP, D), _F32)
    sk8 = jnp.broadcast_to(skip.reshape(1, NT), (8, NT))

    qp, kp, vp = _k0(xp, ntf, Wk, Wq, Wv, Ratt, Rmsg)
    kp2 = kp.reshape(ET * NP, D)
    vp2 = vp.reshape(ET * NP, D)

    aexp, asum_parts = _k1(kp2, qp, meta)
    hacc = _k3(vp2, meta, aexp, zer128)
    asum3 = asum_parts.reshape(NW, NP, H)
    out = _k4(hacc[0], hacc[1], asum3, xp, ntf, Wa, sk8)
    return out[:N]


# speedup vs baseline: 3.7800x; 1.7549x over previous
import math

import jax
import jax.numpy as jnp
from jax import lax
from jax.experimental import pallas as pl
from jax.experimental.pallas import tpu as pltpu
from jax.experimental.pallas import tpu_sc as plsc

N = 10000
E = 320000
D = 128
H = 4
DH = 32
NT = 4
ET = 8

NP = 10240
NC = 2
NS = 16
NW = NC * NS
EW = E // NW
CB = 80
NCH = EW // CB
NGB = CB // 16
RPT = NP // NS
AW = NP * H
AE_W = H * CB
AE_T = NCH * AE_W
BLK = 25
NBK = NCH // BLK

_F32 = jnp.float32
_I32 = jnp.int32

_SC_PARAMS = pltpu.CompilerParams(needs_layout_passes=False)


def _sc_mesh():
    return plsc.VectorSubcoreMesh(core_axis_name="c", subcore_axis_name="s",
                                  num_cores=NC, num_subcores=NS)


def _k0_body(x_ref, nt_ref, wk_ref, wq_ref, wv_ref, ra_ref, rm_ref,
             q_ref, kp_ref, vp_ref):
    x = x_ref[...]
    nt = nt_ref[...]
    k = jnp.zeros_like(x)
    q = jnp.zeros_like(x)
    v = jnp.zeros_like(x)
    for t in range(NT):
        m = (nt == float(t)).astype(_F32)
        k = k + m * jnp.dot(x, wk_ref[t], preferred_element_type=_F32)
        q = q + m * jnp.dot(x, wq_ref[t], preferred_element_type=_F32)
        v = v + m * jnp.dot(x, wv_ref[t], preferred_element_type=_F32)
    q_ref[...] = q
    for e in range(ET):
        kp_ref[e] = jnp.dot(k, ra_ref[e], preferred_element_type=_F32)
        vp_ref[e] = jnp.dot(v, rm_ref[e], preferred_element_type=_F32)


def _k0(xp, ntf, Wk, Wq, Wv, Ratt, Rmsg):
    B = 512
    g = NP // B
    return pl.pallas_call(
        _k0_body,
        grid=(g,),
        in_specs=[
            pl.BlockSpec((B, D), lambda i: (i, 0)),
            pl.BlockSpec((B, 1), lambda i: (i, 0)),
            pl.BlockSpec((NT, D, D), lambda i: (0, 0, 0)),
            pl.BlockSpec((NT, D, D), lambda i: (0, 0, 0)),
            pl.BlockSpec((NT, D, D), lambda i: (0, 0, 0)),
            pl.BlockSpec((ET, D, D), lambda i: (0, 0, 0)),
            pl.BlockSpec((ET, D, D), lambda i: (0, 0, 0)),
        ],
        out_specs=[
            pl.BlockSpec((B, D), lambda i: (i, 0)),
            pl.BlockSpec((ET, B, D), lambda i: (0, i, 0)),
            pl.BlockSpec((ET, B, D), lambda i: (0, i, 0)),
        ],
        out_shape=[
            jax.ShapeDtypeStruct((NP, D), _F32),
            jax.ShapeDtypeStruct((ET, NP, D), _F32),
            jax.ShapeDtypeStruct((ET, NP, D), _F32),
        ],
    )(xp, ntf, Wk, Wq, Wv, Ratt, Rmsg)


def _k1_body(kp_hbm, q_hbm, meta_hbm,
             aexp_hbm, asum_hbm,
             meta0, meta1, kidx0, kidx1, dstc0, dstc1,
             kro0, kro1, qro0, qro1, aexpT, asum1d,
             sm0, sm1, sa0, sa1, sb0, sb1):
    c = lax.axis_index("c")
    s = lax.axis_index("s")
    wid = s * NC + c
    crow = wid * NCH
    META = (meta0, meta1)
    KIDX = (kidx0, kidx1)
    DSTC = (dstc0, dstc1)
    KRO = (kro0, kro1)
    QRO = (qro0, qro1)
    SM = (sm0, sm1)
    SA = (sa0, sa1)
    SB = (sb0, sb1)

    zv = jnp.zeros((16,), _F32)

    def zinit(i, carry):
        asum1d[pl.ds(i * 16, 16)] = zv
        return carry
    lax.fori_loop(0, AW // 16, zinit, 0)

    iota16 = lax.iota(_I32, 16)
    rev4 = (((iota16 & 1) << 3) | ((iota16 & 2) << 1)
            | ((iota16 & 4) >> 1) | ((iota16 & 8) >> 3))

    def fire_meta(p, j):
        jm = jnp.minimum(j, NCH - 1)
        pltpu.async_copy(meta_hbm.at[crow + jm], META[p], SM[p])

    def wait_meta(p):
        pltpu.make_async_copy(meta_hbm.at[crow], META[p], SM[p]).wait()

    def prep(p):
        for g in range(NGB):
            sl = pl.ds(g * 16, 16)
            KIDX[p][sl] = META[p][2, sl] * NP + META[p][0, sl]
            DSTC[p][sl] = META[p][1, sl]
        pltpu.async_copy(kp_hbm.at[KIDX[p]], KRO[p], SA[p])
        pltpu.async_copy(q_hbm.at[DSTC[p]], QRO[p], SB[p])

    def wait_gathers(p):
        pltpu.make_async_copy(kp_hbm.at[KIDX[p]], KRO[p], SA[p]).wait()
        pltpu.make_async_copy(q_hbm.at[DSTC[p]], QRO[p], SB[p]).wait()

    def compute(p, j):
        kro, qro, dstc = KRO[p], QRO[p], DSTC[p]

        def grp(g, carry2):
            dst16 = dstc[pl.ds(g * 16, 16)]
            for h in range(H):
                lo = h * DH
                ps = []
                for e in range(16):
                    r = g * 16 + e
                    ps.append(
                        kro[r, pl.ds(lo, 16)] * qro[r, pl.ds(lo, 16)]
                        + kro[r, pl.ds(lo + 16, 16)]
                        * qro[r, pl.ds(lo + 16, 16)])
                for k in (8, 4, 2, 1):
                    mask = (iota16 & k) == 0
                    ix = iota16 ^ k
                    nxt = []
                    for i in range(0, len(ps), 2):
                        a, b = ps[i], ps[i + 1]
                        ax = a.at[ix].get(mode="promise_in_bounds")
                        bx = b.at[ix].get(mode="promise_in_bounds")
                        nxt.append(jnp.where(mask, a, b)
                                   + jnp.where(mask, ax, bx))
                    ps = nxt
                sv = ps[0].at[rev4].get(mode="promise_in_bounds")
                ae = jnp.exp(sv)
                aexpT[pl.ds(j * AE_W + h * CB + g * 16, 16)] = ae
                plsc.addupdate_scatter(asum1d, [dst16 * H + h], ae)
            return carry2
        lax.fori_loop(0, NGB, grp, 0)

    fire_meta(0, 0)
    fire_meta(1, 1)
    wait_meta(0)
    prep(0)
    fire_meta(0, 2)

    def pair(t, carry):
        j0 = 2 * t
        j1 = j0 + 1
        wait_meta(1)
        prep(1)
        fire_meta(1, j1 + 2)
        wait_gathers(0)
        compute(0, j0)
        wait_meta(0)
        prep(0)
        fire_meta(0, j0 + 4)
        wait_gathers(1)
        compute(1, j1)
        return carry

    lax.fori_loop(0, (NCH - 1) // 2, pair, 0)
    wait_gathers(0)
    compute(0, NCH - 1)
    wait_meta(0)
    wait_meta(1)

    pltpu.sync_copy(aexpT, aexp_hbm.at[pl.ds(wid * AE_T, AE_T)])
    pltpu.sync_copy(asum1d, asum_hbm.at[wid])


def _k1(kp2, qp, meta):
    f = pl.kernel(
        _k1_body,
        out_type=[
            jax.ShapeDtypeStruct((NW * AE_T,), _F32),
            jax.ShapeDtypeStruct((NW, AW), _F32),
        ],
        mesh=_sc_mesh(),
        compiler_params=_SC_PARAMS,
        scratch_types=[
            pltpu.VMEM((3, CB), _I32),
            pltpu.VMEM((3, CB), _I32),
            pltpu.VMEM((CB,), _I32),
            pltpu.VMEM((CB,), _I32),
            pltpu.VMEM((CB,), _I32),
            pltpu.VMEM((CB,), _I32),
            pltpu.VMEM((CB, D), _F32),
            pltpu.VMEM((CB, D), _F32),
            pltpu.VMEM((CB, D), _F32),
            pltpu.VMEM((CB, D), _F32),
            pltpu.VMEM((AE_T,), _F32),
            pltpu.VMEM((AW,), _F32),
            pltpu.SemaphoreType.DMA,
            pltpu.SemaphoreType.DMA,
            pltpu.SemaphoreType.DMA,
            pltpu.SemaphoreType.DMA,
            pltpu.SemaphoreType.DMA,
            pltpu.SemaphoreType.DMA,
        ],
    )
    return f(kp2, qp, meta)


def _k3_body(vp_hbm, meta_hbm, aexp_hbm, zer128_hbm,
             hacc_hbm,
             meta0, meta1, vidx0, vidx1, dstc0, dstc1, dsts0, dsts1,
             vro0, vro1, mrow0, mrow1, ae0, ae1, hacc_sh,
             sm0, sm1, sv0, sv1, sa0, sa1, ss0, ss1):
    c = lax.axis_index("c")
    s = lax.axis_index("s")
    wid = s * NC + c
    row0 = s * RPT
    crow = wid * NCH
    abase = wid * AE_T
    META = (meta0, meta1)
    VIDX = (vidx0, vidx1)
    DSTC = (dstc0, dstc1)
    DSTS = (dsts0, dsts1)
    VRO = (vro0, vro1)
    MROW = (mrow0, mrow1)
    AEC = (ae0, ae1)
    SM = (sm0, sm1)
    SV = (sv0, sv1)
    SA = (sa0, sa1)
    SS = (ss0, ss1)

    pltpu.sync_copy(zer128_hbm.at[pl.ds(row0, RPT)],
                    hacc_sh.at[pl.ds(row0, RPT)])
    plsc.subcore_barrier()

    iota16 = lax.iota(_I32, 16)

    def fire_meta(p, j):
        jm = jnp.minimum(j, NCH - 1)
        pltpu.async_copy(meta_hbm.at[crow + jm], META[p], SM[p])
        pltpu.async_copy(aexp_hbm.at[pl.ds(abase + jm * AE_W, AE_W)],
                         AEC[p], SA[p])

    def wait_meta(p):
        pltpu.make_async_copy(meta_hbm.at[crow], META[p], SM[p]).wait()
        pltpu.make_async_copy(aexp_hbm.at[pl.ds(abase, AE_W)],
                              AEC[p], SA[p]).wait()

    def prep(p):
        for g in range(NGB):
            sl = pl.ds(g * 16, 16)
            VIDX[p][sl] = META[p][2, sl] * NP + META[p][0, sl]
            DSTC[p][sl] = META[p][1, sl]
        pltpu.async_copy(vp_hbm.at[VIDX[p]], VRO[p], SV[p])

    def wait_gather(p):
        pltpu.make_async_copy(vp_hbm.at[VIDX[p]], VRO[p], SV[p]).wait()

    def wait_scatter(p):
        pltpu.make_async_copy(MROW[p], hacc_sh.at[DSTS[p]], SS[p]).wait()

    def compute_fire(p, t, first):
        if first is not None:
            @pl.when(t > first)
            def _():
                wait_scatter(p)
        else:
            wait_scatter(p)
        vro, mrow, aec = VRO[p], MROW[p], AEC[p]
        for g in range(NGB):
            sl = pl.ds(g * 16, 16)
            DSTS[p][sl] = DSTC[p][sl]

        def grp(g, carry2):
            aes = [aec[pl.ds(h * CB + g * 16, 16)] for h in range(H)]
            for e in range(16):
                r = g * 16 + e
                lane = jnp.full((16,), e, _I32)
                for h in range(H):
                    sp = aes[h].at[lane].get(mode="promise_in_bounds")
                    lo = h * DH
                    mrow[r, pl.ds(lo, 16)] = vro[r, pl.ds(lo, 16)] * sp
                    mrow[r, pl.ds(lo + 16, 16)] = (
                        vro[r, pl.ds(lo + 16, 16)] * sp)
            return carry2
        lax.fori_loop(0, NGB, grp, 0)
        pltpu.async_copy(mrow, hacc_sh.at[DSTS[p]], SS[p], add=True)

    fire_meta(0, 0)
    fire_meta(1, 1)
    wait_meta(0)
    prep(0)
    fire_meta(0, 2)

    def pair(t, carry):
        j0 = 2 * t
        wait_meta(1)
        prep(1)
        fire_meta(1, j0 + 3)
        wait_gather(0)
        compute_fire(0, t, 0)
        wait_meta(0)
        prep(0)
        fire_meta(0, j0 + 4)
        wait_gather(1)
        compute_fire(1, t, 0)
        return carry

    lax.fori_loop(0, (NCH - 1) // 2, pair, 0)
    wait_gather(0)
    compute_fire(0, 0, None)
    wait_meta(0)
    wait_meta(1)
    wait_scatter(0)
    wait_scatter(1)
    plsc.subcore_barrier()
    pltpu.sync_copy(hacc_sh.at[pl.ds(row0, RPT)],
                    hacc_hbm.at[c, pl.ds(row0, RPT)])


def _k3(vp2, meta, aexp, zer128):
    f = pl.kernel(
        _k3_body,
        out_type=jax.ShapeDtypeStruct((NC, NP, D), _F32),
        mesh=_sc_mesh(),
        compiler_params=_SC_PARAMS,
        scratch_types=[
            pltpu.VMEM((3, CB), _I32),
            pltpu.VMEM((3, CB), _I32),
            pltpu.VMEM((CB,), _I32),
            pltpu.VMEM((CB,), _I32),
            pltpu.VMEM((CB,), _I32),
            pltpu.VMEM((CB,), _I32),
            pltpu.VMEM((CB,), _I32),
            pltpu.VMEM((CB,), _I32),
            pltpu.VMEM((CB, D), _F32),
            pltpu.VMEM((CB, D), _F32),
            pltpu.VMEM((CB, D), _F32),
            pltpu.VMEM((CB, D), _F32),
            pltpu.VMEM((AE_W,), _F32),
            pltpu.VMEM((AE_W,), _F32),
            pltpu.VMEM_SHARED((NP, D), _F32),
            pltpu.SemaphoreType.DMA,
            pltpu.SemaphoreType.DMA,
            pltpu.SemaphoreType.DMA,
            pltpu.SemaphoreType.DMA,
            pltpu.SemaphoreType.DMA,
            pltpu.SemaphoreType.DMA,
            pltpu.SemaphoreType.DMA,
            pltpu.SemaphoreType.DMA,
        ],
    )
    return f(vp2, meta, aexp, zer128)


def _k4_body(h0_ref, h1_ref, asum_ref, x_ref, nt_ref, wa_ref, sk_ref,
             out_ref):
    hs = h0_ref[...] + h1_ref[...]
    x = x_ref[...]
    nt = nt_ref[...]
    den4 = jnp.maximum(jnp.sum(asum_ref[...], axis=0), 1e-30)
    den = jnp.concatenate(
        [jnp.broadcast_to(den4[:, h:h + 1], (x.shape[0], DH))
         for h in range(H)], axis=1)
    hn = hs / den
    sig = 1.0 / (1.0 + jnp.exp(-sk_ref[...]))
    acc = jnp.zeros_like(x)
    alpha = jnp.zeros_like(nt)
    for t in range(NT):
        m = (nt == float(t)).astype(_F32)
        acc = acc + m * jnp.dot(hn, wa_ref[t], preferred_element_type=_F32)
        alpha = alpha + m * jnp.broadcast_to(sig[0:1, t:t + 1], nt.shape)
    out_ref[...] = acc * alpha + x * (1.0 - alpha)


def _k4(h0, h1, asum3, xp, ntf, Wa, sk8):
    B = 512
    g = NP // B
    return pl.pallas_call(
        _k4_body,
        grid=(g,),
        in_specs=[
            pl.BlockSpec((B, D), lambda i: (i, 0)),
            pl.BlockSpec((B, D), lambda i: (i, 0)),
            pl.BlockSpec((NW, B, H), lambda i: (0, i, 0)),
            pl.BlockSpec((B, D), lambda i: (i, 0)),
            pl.BlockSpec((B, 1), lambda i: (i, 0)),
            pl.BlockSpec((NT, D, D), lambda i: (0, 0, 0)),
            pl.BlockSpec((8, NT), lambda i: (0, 0)),
        ],
        out_specs=pl.BlockSpec((B, D), lambda i: (i, 0)),
        out_shape=jax.ShapeDtypeStruct((NP, D), _F32),
    )(h0, h1, asum3, xp, ntf, Wa, sk8)


def kernel(x_node, edge_index, ntype, etype, Wk, Wq, Wv, Wa,
           rel_att, rel_msg, rel_pri, skip):
    xp = jnp.zeros((NP, D), _F32).at[:N].set(x_node)
    ntf = jnp.zeros((NP, 1), _F32).at[:N, 0].set(ntype.astype(_F32))

    scale = rel_pri / math.sqrt(DH)
    Ratt = jnp.zeros((ET, D, D), _F32)
    Rmsg = jnp.zeros((ET, D, D), _F32)
    for i in range(H):
        blk = slice(i * DH, (i + 1) * DH)
        Ratt = Ratt.at[:, blk, blk].set(rel_att[i] * scale[i][:, None, None])
        Rmsg = Rmsg.at[:, blk, blk].set(rel_msg[i])

    meta = jnp.stack([edge_index[0].reshape(E // CB, CB),
                      edge_index[1].reshape(E // CB, CB),
                      etype.reshape(E // CB, CB)], axis=1)
    zer128 = jnp.zeros((NP, D), _F32)
    sk8 = jnp.broadcast_to(skip.reshape(1, NT), (8, NT))

    qp, kp, vp = _k0(xp, ntf, Wk, Wq, Wv, Ratt, Rmsg)
    kp2 = kp.reshape(ET * NP, D)
    vp2 = vp.reshape(ET * NP, D)

    aexp, asum_parts = _k1(kp2, qp, meta)
    hacc = _k3(vp2, meta, aexp, zer128)
    asum3 = asum_parts.reshape(NW, NP, H)
    out = _k4(hacc[0], hacc[1], asum3, xp, ntf, Wa, sk8)
    return out[:N]
